# Initial kernel scaffold; baseline (speedup 1.0000x reference)
import jax
import jax.numpy as jnp
from jax.experimental import pallas as pl


def kernel(feature, xyz, neigh_idx, encode_list, W1, g1, b1, Wfc, Watt, gatt, batt, Wg1, gg1, bg1, Wg2, gg2, bg2, W2, g2, b2, Ws, gs, bs):
    raise NotImplementedError("write your pallas kernel here")


# speedup vs baseline: 1.3703x; 1.3703x over previous
import functools

import jax
import jax.numpy as jnp
import numpy as np
from jax import lax
from jax.experimental import pallas as pl
from jax.experimental.pallas import tpu as pltpu
from jax.experimental.pallas import tpu_sc as plsc

_EPS = 1e-5
_K = 16
_NT = 256


def _sc_gather(table, idx_flat, n_rows, d, n_inner):
    info = plsc.get_sparse_core_info()
    nc, ns = info.num_cores, info.num_subcores
    nw = nc * ns
    rows_w = n_rows // nw
    n_chunks = rows_w // 128
    n_groups = n_chunks // n_inner
    g_rows = n_inner * 128
    idx3 = idx_flat.reshape(nw, n_chunks, 128)
    mesh = plsc.VectorSubcoreMesh(core_axis_name="c", subcore_axis_name="s")

    @functools.partial(
        pl.kernel,
        mesh=mesh,
        out_type=jax.ShapeDtypeStruct((n_rows, d), jnp.float32),
        scratch_types=[
            pltpu.VMEM((n_chunks, 128), jnp.int32),
            pltpu.VMEM((g_rows, d), jnp.float32),
            pltpu.SemaphoreType.DMA,
        ],
    )
    def gk(table_hbm, idx_hbm, out_hbm, idx_v, rows_v, sem):
        wid = lax.axis_index("s") * nc + lax.axis_index("c")
        base = wid * rows_w
        pltpu.sync_copy(idx_hbm.at[wid], idx_v)

        def group(g, carry):
            cps = [
                pltpu.async_copy(
                    table_hbm.at[idx_v.at[g * n_inner + b]],
                    rows_v.at[pl.ds(b * 128, 128)],
                    sem,
                )
                for b in range(n_inner)
            ]
            for cp in cps:
                cp.wait()
            pltpu.sync_copy(rows_v, out_hbm.at[pl.ds(base + g * g_rows, g_rows)])
            return carry

        lax.fori_loop(0, n_groups, group, 0)

    return gk(table, idx3)


def _tc1_body(feat, xyzp, nbx, w1p, b1r, scen, snb, e0, wfcp, wattp, battp,
              wae, waf, bg1r, wbe, wbf, a_ref, bv_ref, enc_ref):
    nt, k = _NT, _K
    cen_p = xyzp[...][:, :16]
    nb = nbx[...][:, :16]
    cen = jnp.broadcast_to(cen_p[:, None, :], (nt, k, 16)).reshape(nt * k, 16)
    rel = cen - nb
    dist = jnp.sqrt(jnp.sum(rel * rel, axis=-1, keepdims=True) + 1e-12)
    fx = jnp.dot(cen, scen[...]) + jnp.dot(nb, snb[...]) + dist * e0[...]
    logits = jnp.dot(fx, wfcp[...]).reshape(nt, k, 16)
    m = jnp.max(logits, axis=1, keepdims=True)
    e = jnp.exp(logits - m)
    ssum = jnp.sum(e, axis=1, keepdims=True)
    attn = (e / ssum).reshape(nt * k, 16)
    f_agg = jnp.sum((fx * attn).reshape(nt, k, 16), axis=1)
    enc = jnp.maximum(jnp.dot(f_agg, wattp[...]) + battp[...], 0.0)
    f_pc = jnp.maximum(
        lax.dot_general(feat[...], w1p[...], (((0,), (0,)), ((), ()))) + b1r[...],
        0.0)
    a_ref[...] = jnp.dot(enc, wae[...]) + jnp.dot(f_pc, waf[...]) + bg1r[...]
    bv_ref[...] = jnp.dot(enc, wbe[...]) + jnp.dot(f_pc, wbf[...])
    enc_ref[...] = enc


def _tc2_body(g_ref, a_ref, feat_ref, wg2t, bg2r, w2, ws, bsum, out_ref):
    nt, k = _NT, _K
    g = g_ref[...][:, :64]
    a = a_ref[...]
    h = jnp.maximum(g.reshape(nt, k, 64) + a[:, None, :], 0.0).reshape(nt * k, 64)
    h2 = lax.dot_general(h, wg2t[...], (((1,), (0,)), ((), ())))
    mx = jnp.maximum(jnp.max(h2.reshape(nt, k, 128), axis=1) + bg2r[...], 0.0)
    y = (lax.dot_general(mx, w2[...], (((1,), (1,)), ((), ())))
         + lax.dot_general(feat_ref[...], ws[...], (((0,), (1,)), ((), ())))
         + bsum[...])
    out_ref[...] = jnp.maximum(y, 0.2 * y)


def kernel(feature, xyz, neigh_idx, encode_list, W1, g1, b1, Wfc, Watt, gatt,
           batt, Wg1, gg1, bg1, Wg2, gg2, bg2, W2, g2, b2, Ws, gs, bs):
    del encode_list
    B, d_in, N, _ = feature.shape
    k = neigh_idx.shape[-1]
    npad = ((N + _NT - 1) // _NT) * _NT
    ep = npad * k
    s = 1.0 / np.sqrt(1.0 + _EPS)

    feat = feature[0, :, :, 0]
    featp = jnp.pad(feat, ((0, 0), (0, npad - N)))
    xyzp = jnp.pad(xyz[0], ((0, npad - N), (0, 125)))
    idx_flat = jnp.pad(neigh_idx[0], ((0, npad - N), (0, 0))).astype(
        jnp.int32).reshape(ep)

    w1p = (W1 * (g1 * s)[:, None]).T
    b1r = b1[None, :]
    scen_np = np.zeros((16, 16), np.float32)
    snb_np = np.zeros((16, 16), np.float32)
    for c in range(3):
        scen_np[c, 1 + c] = 1.0
        scen_np[c, 4 + c] = 1.0
        snb_np[c, 1 + c] = -1.0
        snb_np[c, 7 + c] = 1.0
    e0_np = np.zeros((1, 16), np.float32)
    e0_np[0, 0] = 1.0
    scen, snb, e0 = jnp.asarray(scen_np), jnp.asarray(snb_np), jnp.asarray(e0_np)
    wfcp = jnp.zeros((16, 16), jnp.float32).at[:10, :10].set(Wfc.T)
    wattf = Watt * (gatt * s)[:, None]
    wattp = jnp.zeros((16, 16), jnp.float32).at[:10, :10].set(wattf.T)
    battp = jnp.zeros((1, 16), jnp.float32).at[0, :10].set(batt)
    wg1f = Wg1 * (gg1 * s)[:, None]
    wa = wg1f[:, :74] - wg1f[:, 74:]
    wb = wg1f[:, 74:]
    wae = jnp.zeros((16, 64), jnp.float32).at[:10, :].set(wa[:, :10].T)
    waf = wa[:, 10:].T
    bg1r = bg1[None, :]
    wbe = jnp.zeros((16, 128), jnp.float32).at[:10, :64].set(wb[:, :10].T)
    wbf = jnp.zeros((64, 128), jnp.float32).at[:, :64].set(wb[:, 10:].T)
    wg2t = (Wg2 * (gg2 * s)[:, None]).T
    bg2r = bg2[None, :]
    w2f = W2 * (g2 * s)[:, None]
    wsf = Ws * (gs * s)[:, None]
    bsum = (b2 + bs)[None, :]

    nxyz = _sc_gather(xyzp, idx_flat, ep, 128, 4)

    grid = npad // _NT
    wspec = lambda shape: pl.BlockSpec(shape, lambda i: (0, 0))
    a_arr, bv_arr, enc_arr = pl.pallas_call(
        _tc1_body,
        grid=(grid,),
        in_specs=[
            pl.BlockSpec((128, _NT), lambda i: (0, i)),
            pl.BlockSpec((_NT, 128), lambda i: (i, 0)),
            pl.BlockSpec((_NT * _K, 128), lambda i: (i, 0)),
            wspec((128, 64)), wspec((1, 64)), wspec((16, 16)), wspec((16, 16)),
            wspec((1, 16)), wspec((16, 16)), wspec((16, 16)), wspec((1, 16)),
            wspec((16, 64)), wspec((64, 64)), wspec((1, 64)), wspec((16, 128)),
            wspec((64, 128)),
        ],
        out_specs=[
            pl.BlockSpec((_NT, 64), lambda i: (i, 0)),
            pl.BlockSpec((_NT, 128), lambda i: (i, 0)),
            pl.BlockSpec((_NT, 16), lambda i: (i, 0)),
        ],
        out_shape=[
            jax.ShapeDtypeStruct((npad, 64), jnp.float32),
            jax.ShapeDtypeStruct((npad, 128), jnp.float32),
            jax.ShapeDtypeStruct((npad, 16), jnp.float32),
        ],
    )(featp, xyzp, nxyz, w1p, b1r, scen, snb, e0, wfcp, wattp, battp,
      wae, waf, bg1r, wbe, wbf)

    gb = _sc_gather(bv_arr, idx_flat, ep, 128, 4)

    out_pm = pl.pallas_call(
        _tc2_body,
        grid=(grid,),
        in_specs=[
            pl.BlockSpec((_NT * _K, 128), lambda i: (i, 0)),
            pl.BlockSpec((_NT, 64), lambda i: (i, 0)),
            pl.BlockSpec((128, _NT), lambda i: (0, i)),
            wspec((64, 128)), wspec((1, 128)), wspec((256, 128)),
            wspec((256, 128)), wspec((1, 256)),
        ],
        out_specs=pl.BlockSpec((_NT, 256), lambda i: (i, 0)),
        out_shape=jax.ShapeDtypeStruct((npad, 256), jnp.float32),
    )(gb, a_arr, featp, wg2t, bg2r, w2f, wsf, bsum)

    out = out_pm[:N].T[None, :, :, None]
    enc_out = enc_arr[:N, :10].T[None, :, :, None]
    return out, enc_out


# speedup vs baseline: 1.3979x; 1.0202x over previous
import functools

import jax
import jax.numpy as jnp
import numpy as np
from jax import lax
from jax.experimental import pallas as pl
from jax.experimental.pallas import tpu as pltpu
from jax.experimental.pallas import tpu_sc as plsc

_EPS = 1e-5
_K = 16
_NT = 256


def _sc_gather(table, idx_flat, n_rows, d):
    info = plsc.get_sparse_core_info()
    nc, ns = info.num_cores, info.num_subcores
    nw = nc * ns
    rows_w = n_rows // nw
    n_chunks = rows_w // 128
    n_steps = n_chunks // 2
    idx3 = idx_flat.reshape(nw, n_chunks, 128)
    mesh = plsc.VectorSubcoreMesh(core_axis_name="c", subcore_axis_name="s")

    @functools.partial(
        pl.kernel,
        mesh=mesh,
        out_type=jax.ShapeDtypeStruct((n_rows, d), jnp.float32),
        scratch_types=[
            pltpu.VMEM((n_chunks, 128), jnp.int32),
            pltpu.VMEM((2, 128, d), jnp.float32),
            pltpu.SemaphoreType.DMA,
            pltpu.SemaphoreType.DMA,
            pltpu.SemaphoreType.DMA,
            pltpu.SemaphoreType.DMA,
        ],
    )
    def gk(table_hbm, idx_hbm, out_hbm, idx_v, buf_v, gs0, gs1, ss0, ss1):
        wid = lax.axis_index("s") * nc + lax.axis_index("c")
        base = wid * rows_w
        pltpu.sync_copy(idx_hbm.at[wid], idx_v)
        gsems = (gs0, gs1)
        ssems = (ss0, ss1)

        def step(s, carry):
            gcps = []
            for b in range(2):
                c = 2 * s + b

                @pl.when(s > 0)
                def _drain():
                    pltpu.make_async_copy(
                        buf_v.at[b],
                        out_hbm.at[pl.ds(base + c * 128, 128)],
                        ssems[b],
                    ).wait()

                gcps.append(pltpu.async_copy(
                    table_hbm.at[idx_v.at[c]], buf_v.at[b], gsems[b]))
            for b in range(2):
                c = 2 * s + b
                gcps[b].wait()
                pltpu.async_copy(
                    buf_v.at[b], out_hbm.at[pl.ds(base + c * 128, 128)],
                    ssems[b])
            return carry

        lax.fori_loop(0, n_steps, step, 0)
        for b in range(2):
            pltpu.make_async_copy(
                buf_v.at[b], out_hbm.at[pl.ds(base, 128)], ssems[b]).wait()

    return gk(table, idx3)


def _tc1_body(feat, xyzp, nbx, w1p, b1r, scen, snb, e0, wfcp, wattp, battp,
              wae, waf, bg1r, wbe, wbf, a_ref, bv_ref, enc_ref):
    nt, k = _NT, _K
    cen_p = xyzp[...][:, :16]
    nb = nbx[...][:, :16]
    cen = jnp.broadcast_to(cen_p[:, None, :], (nt, k, 16)).reshape(nt * k, 16)
    rel = cen - nb
    dist = jnp.sqrt(jnp.sum(rel * rel, axis=-1, keepdims=True) + 1e-12)
    fx = jnp.dot(cen, scen[...]) + jnp.dot(nb, snb[...]) + dist * e0[...]
    logits = jnp.dot(fx, wfcp[...]).reshape(nt, k, 16)
    m = jnp.max(logits, axis=1, keepdims=True)
    e = jnp.exp(logits - m)
    ssum = jnp.sum(e, axis=1, keepdims=True)
    attn = (e / ssum).reshape(nt * k, 16)
    f_agg = jnp.sum((fx * attn).reshape(nt, k, 16), axis=1)
    enc = jnp.maximum(jnp.dot(f_agg, wattp[...]) + battp[...], 0.0)
    f_pc = jnp.maximum(
        lax.dot_general(feat[...], w1p[...], (((0,), (0,)), ((), ()))) + b1r[...],
        0.0)
    a_ref[...] = jnp.dot(enc, wae[...]) + jnp.dot(f_pc, waf[...]) + bg1r[...]
    bv_ref[...] = jnp.dot(enc, wbe[...]) + jnp.dot(f_pc, wbf[...])
    enc_ref[...] = enc


def _tc2_body(g_ref, a_ref, feat_ref, wg2t, bg2r, w2, ws, bsum, out_ref):
    nt, k = _NT, _K
    g = g_ref[...][:, :64]
    a = a_ref[...]
    h = jnp.maximum(g.reshape(nt, k, 64) + a[:, None, :], 0.0).reshape(nt * k, 64)
    h2 = lax.dot_general(h, wg2t[...], (((1,), (0,)), ((), ())))
    mx = jnp.maximum(jnp.max(h2.reshape(nt, k, 128), axis=1) + bg2r[...], 0.0)
    y = (lax.dot_general(mx, w2[...], (((1,), (1,)), ((), ())))
         + lax.dot_general(feat_ref[...], ws[...], (((0,), (1,)), ((), ())))
         + bsum[...])
    out_ref[...] = jnp.maximum(y, 0.2 * y)


def kernel(feature, xyz, neigh_idx, encode_list, W1, g1, b1, Wfc, Watt, gatt,
           batt, Wg1, gg1, bg1, Wg2, gg2, bg2, W2, g2, b2, Ws, gs, bs):
    del encode_list
    B, d_in, N, _ = feature.shape
    k = neigh_idx.shape[-1]
    npad = ((N + _NT - 1) // _NT) * _NT
    ep = npad * k
    s = 1.0 / np.sqrt(1.0 + _EPS)

    feat = feature[0, :, :, 0]
    featp = jnp.pad(feat, ((0, 0), (0, npad - N)))
    xyzp = jnp.pad(xyz[0], ((0, npad - N), (0, 125)))
    idx_flat = jnp.pad(neigh_idx[0], ((0, npad - N), (0, 0))).astype(
        jnp.int32).reshape(ep)

    w1p = (W1 * (g1 * s)[:, None]).T
    b1r = b1[None, :]
    scen_np = np.zeros((16, 16), np.float32)
    snb_np = np.zeros((16, 16), np.float32)
    for c in range(3):
        scen_np[c, 1 + c] = 1.0
        scen_np[c, 4 + c] = 1.0
        snb_np[c, 1 + c] = -1.0
        snb_np[c, 7 + c] = 1.0
    e0_np = np.zeros((1, 16), np.float32)
    e0_np[0, 0] = 1.0
    scen, snb, e0 = jnp.asarray(scen_np), jnp.asarray(snb_np), jnp.asarray(e0_np)
    wfcp = jnp.zeros((16, 16), jnp.float32).at[:10, :10].set(Wfc.T)
    wattf = Watt * (gatt * s)[:, None]
    wattp = jnp.zeros((16, 16), jnp.float32).at[:10, :10].set(wattf.T)
    battp = jnp.zeros((1, 16), jnp.float32).at[0, :10].set(batt)
    wg1f = Wg1 * (gg1 * s)[:, None]
    wa = wg1f[:, :74] - wg1f[:, 74:]
    wb = wg1f[:, 74:]
    wae = jnp.zeros((16, 64), jnp.float32).at[:10, :].set(wa[:, :10].T)
    waf = wa[:, 10:].T
    bg1r = bg1[None, :]
    wbe = jnp.zeros((16, 128), jnp.float32).at[:10, :64].set(wb[:, :10].T)
    wbf = jnp.zeros((64, 128), jnp.float32).at[:, :64].set(wb[:, 10:].T)
    wg2t = (Wg2 * (gg2 * s)[:, None]).T
    bg2r = bg2[None, :]
    w2f = W2 * (g2 * s)[:, None]
    wsf = Ws * (gs * s)[:, None]
    bsum = (b2 + bs)[None, :]

    nxyz = _sc_gather(xyzp, idx_flat, ep, 128)

    grid = npad // _NT
    wspec = lambda shape: pl.BlockSpec(shape, lambda i: (0, 0))
    a_arr, bv_arr, enc_arr = pl.pallas_call(
        _tc1_body,
        grid=(grid,),
        in_specs=[
            pl.BlockSpec((128, _NT), lambda i: (0, i)),
            pl.BlockSpec((_NT, 128), lambda i: (i, 0)),
            pl.BlockSpec((_NT * _K, 128), lambda i: (i, 0)),
            wspec((128, 64)), wspec((1, 64)), wspec((16, 16)), wspec((16, 16)),
            wspec((1, 16)), wspec((16, 16)), wspec((16, 16)), wspec((1, 16)),
            wspec((16, 64)), wspec((64, 64)), wspec((1, 64)), wspec((16, 128)),
            wspec((64, 128)),
        ],
        out_specs=[
            pl.BlockSpec((_NT, 64), lambda i: (i, 0)),
            pl.BlockSpec((_NT, 128), lambda i: (i, 0)),
            pl.BlockSpec((_NT, 16), lambda i: (i, 0)),
        ],
        out_shape=[
            jax.ShapeDtypeStruct((npad, 64), jnp.float32),
            jax.ShapeDtypeStruct((npad, 128), jnp.float32),
            jax.ShapeDtypeStruct((npad, 16), jnp.float32),
        ],
    )(featp, xyzp, nxyz, w1p, b1r, scen, snb, e0, wfcp, wattp, battp,
      wae, waf, bg1r, wbe, wbf)

    gb = _sc_gather(bv_arr, idx_flat, ep, 128)

    out_pm = pl.pallas_call(
        _tc2_body,
        grid=(grid,),
        in_specs=[
            pl.BlockSpec((_NT * _K, 128), lambda i: (i, 0)),
            pl.BlockSpec((_NT, 64), lambda i: (i, 0)),
            pl.BlockSpec((128, _NT), lambda i: (0, i)),
            wspec((64, 128)), wspec((1, 128)), wspec((256, 128)),
            wspec((256, 128)), wspec((1, 256)),
        ],
        out_specs=pl.BlockSpec((_NT, 256), lambda i: (i, 0)),
        out_shape=jax.ShapeDtypeStruct((npad, 256), jnp.float32),
    )(gb, a_arr, featp, wg2t, bg2r, w2f, wsf, bsum)

    out = out_pm[:N].T[None, :, :, None]
    enc_out = enc_arr[:N, :10].T[None, :, :, None]
    return out, enc_out
